# baseline (speedup 1.0000x reference)
import functools
import math

import jax
import jax.numpy as jnp
from jax import lax
from jax.experimental import pallas as pl
from jax.experimental.pallas import tpu as pltpu
from jax.experimental.pallas import tpu_sc as plsc

ENTITY_NUM = 100000
RELATION_NUM = 1000
DIM = 64
BATCH = 16384
PATHS_PER_PAIR = 4
PATH_LEN = 3
GAMMA = 1.0

NC = 2
NS = 16
L = 16
NW = NC * NS
W = BATCH // NW
C = 32
NCHUNK = W // C
CP = C * PATHS_PER_PAIR
CT = C * PATHS_PER_PAIR * PATH_LEN

_ROWBUFS = ("ehb", "thb", "etb", "ttb", "nehb", "nthb", "netb", "nttb")
_IDXBUFS = ("phv", "ptv", "nhv", "ntv")


def _fast_sqrt(s):
    x = jnp.maximum(s, 1e-30)
    i = plsc.bitcast(x, jnp.int32)
    i = jnp.full((L,), 0x5F3759DF, jnp.int32) - lax.shift_right_logical(i, 1)
    y = plsc.bitcast(i, jnp.float32)
    half = 0.5 * x
    for _ in range(3):
        y = y * (1.5 - half * y * y)
    return x * y


def _body(ent_hbm, rel_hbm, typ_hbm, probs_hbm,
          ph_hbm, pr_hbm, pt_hbm, nh_hbm, nr_hbm, nt_hbm, tok_hbm,
          out_hbm, relv, accv, *sbufs):
    cid = lax.axis_index("c")
    sid = lax.axis_index("s")
    wid = sid * NC + cid

    names = _ROWBUFS + _IDXBUFS + ("prv", "nrv", "tokv", "probv", "sem")
    n = len(names)
    sets = [dict(zip(names, sbufs[:n])), dict(zip(names, sbufs[n:]))]

    accv[...] = jnp.zeros((L,), jnp.float32)
    pltpu.sync_copy(rel_hbm, relv)

    def dma_pairs(base, b):
        return [
            (pr_hbm.at[pl.ds(base, C)], b["prv"]),
            (nr_hbm.at[pl.ds(base, C)], b["nrv"]),
            (tok_hbm.at[pl.ds(base * 12, CT)], b["tokv"]),
            (probs_hbm.at[pl.ds(base * 4, CP)], b["probv"]),
            (ent_hbm.at[b["phv"]], b["ehb"]),
            (typ_hbm.at[b["phv"]], b["thb"]),
            (ent_hbm.at[b["ptv"]], b["etb"]),
            (typ_hbm.at[b["ptv"]], b["ttb"]),
            (ent_hbm.at[b["nhv"]], b["nehb"]),
            (typ_hbm.at[b["nhv"]], b["nthb"]),
            (ent_hbm.at[b["ntv"]], b["netb"]),
            (typ_hbm.at[b["ntv"]], b["nttb"]),
        ]

    def issue(j, b):
        base = wid * W + j * C
        pltpu.sync_copy(ph_hbm.at[pl.ds(base, C)], b["phv"])
        pltpu.sync_copy(pt_hbm.at[pl.ds(base, C)], b["ptv"])
        pltpu.sync_copy(nh_hbm.at[pl.ds(base, C)], b["nhv"])
        pltpu.sync_copy(nt_hbm.at[pl.ds(base, C)], b["ntv"])
        for s, d in dma_pairs(base, b):
            pltpu.async_copy(s, d, b["sem"])

    def drain(j, b):
        base = wid * W + j * C
        for s, d in dma_pairs(base, b):
            pltpu.make_async_copy(s, d, b["sem"]).wait()

    def compute(b):
        def group_body(g, loss16):
            lane = lax.iota(jnp.int32, 16)
            rl = lane + g * L
            rl4 = rl * 4
            rl12 = rl * 12
            pr0 = plsc.load_gather(b["probv"], [rl4])
            pr1 = plsc.load_gather(b["probv"], [rl4 + 1])
            pr2 = plsc.load_gather(b["probv"], [rl4 + 2])
            pr3 = plsc.load_gather(b["probv"], [rl4 + 3])
            pridx = plsc.load_gather(b["prv"], [rl])
            nridx = plsc.load_gather(b["nrv"], [rl])
            trow = [plsc.load_gather(b["tokv"], [rl12 + k]) for k in range(12)]

            z = jnp.zeros((L,), jnp.float32)

            @plsc.parallel_loop(0, DIM, 1, unroll=4, carry=(z, z))
            def c_loop(c, carry):
                s_pos, s_neg = carry
                cv = jnp.bitwise_and(c + lane, DIM - 1)
                eh = plsc.load_gather(b["ehb"], [rl, cv])
                th = plsc.load_gather(b["thb"], [rl, cv])
                et = plsc.load_gather(b["etb"], [rl, cv])
                tt = plsc.load_gather(b["ttb"], [rl, cv])
                neh = plsc.load_gather(b["nehb"], [rl, cv])
                nth = plsc.load_gather(b["nthb"], [rl, cv])
                net = plsc.load_gather(b["netb"], [rl, cv])
                ntt = plsc.load_gather(b["nttb"], [rl, cv])
                rp = plsc.load_gather(relv, [pridx, cv])
                nr = plsc.load_gather(relv, [nridx, cv])
                t = [plsc.load_gather(relv, [trow[k], cv]) for k in range(12)]
                s0 = t[0] + t[1] + t[2]
                s1 = t[3] + t[4] + t[5]
                s2 = t[6] + t[7] + t[8]
                s3 = t[9] + t[10] + t[11]
                pf = pr0 * s0 + pr1 * s1 + pr2 * s2 + pr3 * s3
                pos = eh * th + rp + pf - et * tt
                neg = neh * nth + nr - net * ntt
                return s_pos + pos * pos, s_neg + neg * neg

            s_pos, s_neg = c_loop
            pn = _fast_sqrt(s_pos)
            nn = _fast_sqrt(s_neg)
            dd = GAMMA + pn - nn
            return loss16 + jnp.maximum(dd, 0.0) + 0.001 * (pn + nn)

        loss16 = lax.fori_loop(0, C // L, group_body,
                               jnp.zeros((L,), jnp.float32))
        accv[...] = accv[...] + loss16

    issue(0, sets[0])

    def body2(k, _):
        issue(2 * k + 1, sets[1])
        drain(2 * k, sets[0])
        compute(sets[0])

        @pl.when(k < NCHUNK // 2 - 1)
        def _issue_next():
            issue(2 * k + 2, sets[0])

        drain(2 * k + 1, sets[1])
        compute(sets[1])
        return 0

    lax.fori_loop(0, NCHUNK // 2, body2, 0)
    pltpu.sync_copy(accv, out_hbm.at[pl.ds(wid * L, L)])


@jax.jit
def _run(entity_emb, relation_emb, type_emb, path_probs,
         pos_head, pos_relation, pos_tail,
         neg_head, neg_relation, neg_tail, path_rel_idx):
    mesh = plsc.VectorSubcoreMesh(core_axis_name="c", subcore_axis_name="s",
                                  num_cores=NC, num_subcores=NS)
    one_set = (
        [pltpu.VMEM((C, DIM), jnp.float32)] * len(_ROWBUFS)
        + [pltpu.VMEM((C,), jnp.int32)] * len(_IDXBUFS)
        + [pltpu.VMEM((C,), jnp.int32),
           pltpu.VMEM((C,), jnp.int32),
           pltpu.VMEM((CT,), jnp.int32),
           pltpu.VMEM((CP,), jnp.float32),
           pltpu.SemaphoreType.DMA]
    )
    kern = pl.kernel(
        _body,
        out_type=jax.ShapeDtypeStruct((NW * L,), jnp.float32),
        mesh=mesh,
        compiler_params=pltpu.CompilerParams(
            needs_layout_passes=False, use_tc_tiling_on_sc=False),
        scratch_types=(
            [pltpu.VMEM((RELATION_NUM, DIM), jnp.float32),
             pltpu.VMEM((L,), jnp.float32)]
            + one_set + one_set
        ),
    )
    partials = kern(entity_emb, relation_emb, type_emb, path_probs,
                    pos_head, pos_relation, pos_tail,
                    neg_head, neg_relation, neg_tail, path_rel_idx)
    return jnp.sum(partials)


def kernel(entity_emb, relation_emb, type_emb, path_probs,
           pos_head, pos_relation, pos_tail,
           neg_head, neg_relation, neg_tail, path_rel_idx):
    return _run(entity_emb, relation_emb, type_emb, path_probs,
                pos_head.astype(jnp.int32), pos_relation.astype(jnp.int32),
                pos_tail.astype(jnp.int32), neg_head.astype(jnp.int32),
                neg_relation.astype(jnp.int32), neg_tail.astype(jnp.int32),
                path_rel_idx.astype(jnp.int32))
